# baseline (speedup 1.0000x reference)
import functools

import jax
import jax.numpy as jnp
import numpy as np
from jax import lax
from jax.experimental import pallas as pl
from jax.experimental.pallas import tpu as pltpu
from jax.experimental.pallas import tpu_sc as plsc

N_NODES = 10000
N_EDGES = 320000
D_IN = 128
D_OUT = 128
HALF = 64
HWORDS = HALF // 2

NC = 2
NS = 16
EDGES_PER_SUBCORE = N_EDGES // NS
CH = 80
NCHUNK = EDGES_PER_SUBCORE // CH
N_PAD = 10240
ROWS_PER_TILE = N_PAD // NS

_PERM_HALF = []
for _j in range(HALF // 32):
    _PERM_HALF += [32 * _j + 2 * _t for _t in range(16)]
    _PERM_HALF += [32 * _j + 2 * _t + 1 for _t in range(16)]
_COL_ORDER = np.array(_PERM_HALF + [HALF + p for p in _PERM_HALF])


def _mm_body(a_ref, b_ref, w_ref, o_ref):
    o_ref[...] = (
        jnp.dot(a_ref[0], w_ref[:HALF, :], preferred_element_type=jnp.float32)
        + jnp.dot(b_ref[0], w_ref[HALF:, :], preferred_element_type=jnp.float32)
    )


def _matmul(agg2, Wp):
    R = 1000
    return pl.pallas_call(
        _mm_body,
        grid=(N_NODES // R,),
        in_specs=[
            pl.BlockSpec((1, R, HALF), lambda i: (0, i, 0)),
            pl.BlockSpec((1, R, HALF), lambda i: (1, i, 0)),
            pl.BlockSpec((D_IN, D_OUT), lambda i: (0, 0)),
        ],
        out_specs=pl.BlockSpec((R, D_OUT), lambda i: (i, 0)),
        out_shape=jax.ShapeDtypeStruct((N_NODES, D_OUT), jnp.float32),
    )(agg2, agg2, Wp)


@functools.partial(
    pl.kernel,
    mesh=plsc.VectorSubcoreMesh(core_axis_name="c", subcore_axis_name="s"),
    out_type=jax.ShapeDtypeStruct((NC, N_PAD, HALF), jnp.float32),
    scratch_types=[
        pltpu.VMEM((NCHUNK, CH), jnp.int32),
        pltpu.VMEM((NCHUNK, CH), jnp.int32),
        pltpu.VMEM((NCHUNK, CH), jnp.float32),
        pltpu.VMEM((CH, HWORDS), jnp.int32),
        pltpu.VMEM((CH, HWORDS), jnp.int32),
        pltpu.VMEM((CH, HALF), jnp.float32),
        pltpu.VMEM((CH, HALF), jnp.float32),
        pltpu.VMEM_SHARED((N_PAD, HALF), jnp.float32),
        pltpu.SemaphoreType.DMA,
        pltpu.SemaphoreType.DMA,
        pltpu.SemaphoreType.DMA,
        pltpu.SemaphoreType.DMA,
    ],
    compiler_params=pltpu.CompilerParams(use_tc_tiling_on_sc=False),
)
def _sc_agg(x_lo, x_hi, src_h, dst_h, wgt_h, zero_h, out_h,
            src_v, dst_v, wgt_v, graw0_v, graw1_v, sbuf0_v, sbuf1_v,
            acc_s, gsem0, gsem1, ssem0, ssem1):
    c = lax.axis_index("c")
    s = lax.axis_index("s")

    pltpu.sync_copy(zero_h, acc_s.at[pl.ds(s * ROWS_PER_TILE, ROWS_PER_TILE)])
    pltpu.sync_copy(src_h.at[s], src_v)
    pltpu.sync_copy(dst_h.at[s], dst_v)
    pltpu.sync_copy(wgt_h.at[s], wgt_v)
    plsc.subcore_barrier()

    hi_mask = jnp.full((16,), -65536, jnp.int32)

    def run_edges(x_h):
        bufs = ((graw0_v, sbuf0_v, gsem0, ssem0),
                (graw1_v, sbuf1_v, gsem1, ssem1))

        pltpu.async_copy(x_h.at[src_v.at[0]], graw0_v, gsem0)
        pltpu.async_copy(x_h.at[src_v.at[1]], graw1_v, gsem1)

        def do_pair(i, carry):
            for b in range(2):
                graw_v, sbuf_v, gsem, ssem = bufs[b]
                k = 2 * i + b
                pltpu.make_async_copy(x_h.at[src_v.at[k]], graw_v, gsem).wait()

                @pl.when(i >= 1)
                def _(sbuf_v=sbuf_v, ssem=ssem):
                    pltpu.make_async_copy(
                        sbuf_v, acc_s.at[dst_v.at[0]], ssem).wait()

                def mul_grp(g, carry2, graw_v=graw_v, sbuf_v=sbuf_v, k=k):
                    wv = wgt_v[k, pl.ds(g * 16, 16)]
                    for t in range(16):
                        e = g * 16 + t
                        w = wv[t]
                        for j in range(HWORDS // 16):
                            v32 = graw_v[e, pl.ds(j * 16, 16)]
                            even = lax.bitcast_convert_type(
                                v32 << 16, jnp.float32)
                            odd = lax.bitcast_convert_type(
                                v32 & hi_mask, jnp.float32)
                            sbuf_v[e, pl.ds(j * 32, 16)] = even * w
                            sbuf_v[e, pl.ds(j * 32 + 16, 16)] = odd * w
                    return carry2

                lax.fori_loop(0, CH // 16, mul_grp, 0)

                @pl.when(i < NCHUNK // 2 - 1)
                def _(graw_v=graw_v, gsem=gsem, k=k):
                    pltpu.async_copy(x_h.at[src_v.at[k + 2]], graw_v, gsem)

                pltpu.async_copy(sbuf_v, acc_s.at[dst_v.at[k]], ssem, add=True)

            return carry

        lax.fori_loop(0, NCHUNK // 2, do_pair, 0)

        pltpu.make_async_copy(sbuf0_v, acc_s.at[dst_v.at[0]], ssem0).wait()
        pltpu.make_async_copy(sbuf1_v, acc_s.at[dst_v.at[0]], ssem1).wait()

    @pl.when(c == 0)
    def _():
        run_edges(x_lo)

    @pl.when(c == 1)
    def _():
        run_edges(x_hi)

    plsc.subcore_barrier()
    pltpu.sync_copy(
        acc_s.at[pl.ds(s * ROWS_PER_TILE, ROWS_PER_TILE)],
        out_h.at[c, pl.ds(s * ROWS_PER_TILE, ROWS_PER_TILE)],
    )


def kernel(x, edge_index, edge_weight, W):
    src = edge_index[0].astype(jnp.int32)
    dst = edge_index[1].astype(jnp.int32)
    xb = x.astype(jnp.bfloat16)
    x_lo = lax.bitcast_convert_type(
        xb[:, :HALF].reshape(N_NODES, HWORDS, 2), jnp.int32)
    x_hi = lax.bitcast_convert_type(
        xb[:, HALF:].reshape(N_NODES, HWORDS, 2), jnp.int32)
    src3 = src.reshape(NS, NCHUNK, CH)
    dst3 = dst.reshape(NS, NCHUNK, CH)
    wgt3 = edge_weight.reshape(NS, NCHUNK, CH)
    zeros = jnp.zeros((ROWS_PER_TILE, HALF), jnp.float32)
    agg2 = _sc_agg(x_lo, x_hi, src3, dst3, wgt3, zeros)
    Wp = W[jnp.asarray(_COL_ORDER), :]
    return _matmul(agg2, Wp)


# speedup vs baseline: 1.5575x; 1.5575x over previous
import functools

import jax
import jax.numpy as jnp
from jax import lax
from jax.experimental import pallas as pl
from jax.experimental.pallas import tpu as pltpu
from jax.experimental.pallas import tpu_sc as plsc

N_NODES = 10000
N_EDGES = 320000
D_IN = 128
D_OUT = 128
HALF = 64

NC = 2
NS = 16
EDGES_PER_SUBCORE = N_EDGES // NS
CH = 80
NCHUNK = EDGES_PER_SUBCORE // CH
N_PAD = 10240
ROWS_PER_TILE = N_PAD // NS


def _mm_body(a_ref, b_ref, w_ref, o_ref):
    o_ref[...] = (
        jnp.dot(a_ref[0], w_ref[:HALF, :], preferred_element_type=jnp.float32)
        + jnp.dot(b_ref[0], w_ref[HALF:, :], preferred_element_type=jnp.float32)
    )


def _matmul(agg2, W):
    R = 1000
    return pl.pallas_call(
        _mm_body,
        grid=(N_NODES // R,),
        in_specs=[
            pl.BlockSpec((1, R, HALF), lambda i: (0, i, 0)),
            pl.BlockSpec((1, R, HALF), lambda i: (1, i, 0)),
            pl.BlockSpec((D_IN, D_OUT), lambda i: (0, 0)),
        ],
        out_specs=pl.BlockSpec((R, D_OUT), lambda i: (i, 0)),
        out_shape=jax.ShapeDtypeStruct((N_NODES, D_OUT), jnp.float32),
    )(agg2, agg2, W)


@functools.partial(
    pl.kernel,
    mesh=plsc.VectorSubcoreMesh(core_axis_name="c", subcore_axis_name="s"),
    out_type=jax.ShapeDtypeStruct((NC, N_PAD, HALF), jnp.float32),
    scratch_types=[
        pltpu.VMEM((NCHUNK, CH), jnp.int32),
        pltpu.VMEM((NCHUNK, CH), jnp.int32),
        pltpu.VMEM((NCHUNK, CH), jnp.float32),
        pltpu.VMEM((CH, HALF), jnp.float32),
        pltpu.VMEM((CH, HALF), jnp.float32),
        pltpu.VMEM((CH, HALF), jnp.float32),
        pltpu.VMEM((CH, HALF), jnp.float32),
        pltpu.VMEM_SHARED((N_PAD, HALF), jnp.float32),
        pltpu.SemaphoreType.DMA,
        pltpu.SemaphoreType.DMA,
        pltpu.SemaphoreType.DMA,
        pltpu.SemaphoreType.DMA,
    ],
    compiler_params=pltpu.CompilerParams(use_tc_tiling_on_sc=False),
)
def _sc_agg(x_lo, x_hi, src_h, dst_h, wgt_h, zero_h, out_h,
            src_v, dst_v, wgt_v, graw0_v, graw1_v, sbuf0_v, sbuf1_v,
            acc_s, gsem0, gsem1, ssem0, ssem1):
    c = lax.axis_index("c")
    s = lax.axis_index("s")

    pltpu.sync_copy(zero_h, acc_s.at[pl.ds(s * ROWS_PER_TILE, ROWS_PER_TILE)])
    pltpu.sync_copy(src_h.at[s], src_v)
    pltpu.sync_copy(dst_h.at[s], dst_v)
    pltpu.sync_copy(wgt_h.at[s], wgt_v)
    plsc.subcore_barrier()

    def run_edges(x_h):
        bufs = ((graw0_v, sbuf0_v, gsem0, ssem0),
                (graw1_v, sbuf1_v, gsem1, ssem1))

        pltpu.async_copy(x_h.at[src_v.at[0]], graw0_v, gsem0)
        pltpu.async_copy(x_h.at[src_v.at[1]], graw1_v, gsem1)

        def do_pair(i, carry):
            for b in range(2):
                graw_v, sbuf_v, gsem, ssem = bufs[b]
                k = 2 * i + b
                pltpu.make_async_copy(x_h.at[src_v.at[k]], graw_v, gsem).wait()

                @pl.when(i >= 1)
                def _(sbuf_v=sbuf_v, ssem=ssem):
                    pltpu.make_async_copy(
                        sbuf_v, acc_s.at[dst_v.at[0]], ssem).wait()

                def mul_grp(g, carry2, graw_v=graw_v, sbuf_v=sbuf_v, k=k):
                    wv = wgt_v[k, pl.ds(g * 16, 16)]
                    for t in range(16):
                        e = g * 16 + t
                        w = wv[t]
                        for j in range(HALF // 16):
                            sl = pl.ds(j * 16, 16)
                            sbuf_v[e, sl] = graw_v[e, sl] * w
                    return carry2

                lax.fori_loop(0, CH // 16, mul_grp, 0)

                @pl.when(i < NCHUNK // 2 - 1)
                def _(graw_v=graw_v, gsem=gsem, k=k):
                    pltpu.async_copy(x_h.at[src_v.at[k + 2]], graw_v, gsem)

                pltpu.async_copy(sbuf_v, acc_s.at[dst_v.at[k]], ssem, add=True)

            return carry

        lax.fori_loop(0, NCHUNK // 2, do_pair, 0)

        pltpu.make_async_copy(sbuf0_v, acc_s.at[dst_v.at[0]], ssem0).wait()
        pltpu.make_async_copy(sbuf1_v, acc_s.at[dst_v.at[0]], ssem1).wait()

    @pl.when(c == 0)
    def _():
        run_edges(x_lo)

    @pl.when(c == 1)
    def _():
        run_edges(x_hi)

    plsc.subcore_barrier()
    pltpu.sync_copy(
        acc_s.at[pl.ds(s * ROWS_PER_TILE, ROWS_PER_TILE)],
        out_h.at[c, pl.ds(s * ROWS_PER_TILE, ROWS_PER_TILE)],
    )


def kernel(x, edge_index, edge_weight, W):
    src = edge_index[0].astype(jnp.int32)
    dst = edge_index[1].astype(jnp.int32)
    x_lo = x[:, :HALF]
    x_hi = x[:, HALF:]
    src3 = src.reshape(NS, NCHUNK, CH)
    dst3 = dst.reshape(NS, NCHUNK, CH)
    wgt3 = edge_weight.reshape(NS, NCHUNK, CH)
    zeros = jnp.zeros((ROWS_PER_TILE, HALF), jnp.float32)
    agg2 = _sc_agg(x_lo, x_hi, src3, dst3, wgt3, zeros)
    return _matmul(agg2, W)


# speedup vs baseline: 1.6085x; 1.0327x over previous
import functools

import jax
import jax.numpy as jnp
from jax import lax
from jax.experimental import pallas as pl
from jax.experimental.pallas import tpu as pltpu
from jax.experimental.pallas import tpu_sc as plsc

N_NODES = 10000
N_EDGES = 320000
D_IN = 128
D_OUT = 128
HALF = 64

NC = 2
NS = 16
EDGES_PER_SUBCORE = N_EDGES // NS
CH = 80
NCHUNK = EDGES_PER_SUBCORE // CH
N_PAD = 10240
ROWS_PER_TILE = N_PAD // NS


def _mm_body(a_ref, b_ref, w_ref, o_ref):
    o_ref[...] = (
        jnp.dot(a_ref[0], w_ref[:HALF, :], preferred_element_type=jnp.float32)
        + jnp.dot(b_ref[0], w_ref[HALF:, :], preferred_element_type=jnp.float32)
    )


def _matmul(agg2, W):
    R = 1000
    return pl.pallas_call(
        _mm_body,
        grid=(N_NODES // R,),
        in_specs=[
            pl.BlockSpec((1, R, HALF), lambda i: (0, i, 0)),
            pl.BlockSpec((1, R, HALF), lambda i: (1, i, 0)),
            pl.BlockSpec((D_IN, D_OUT), lambda i: (0, 0)),
        ],
        out_specs=pl.BlockSpec((R, D_OUT), lambda i: (i, 0)),
        out_shape=jax.ShapeDtypeStruct((N_NODES, D_OUT), jnp.float32),
    )(agg2, agg2, W)


@functools.partial(
    pl.kernel,
    mesh=plsc.VectorSubcoreMesh(core_axis_name="c", subcore_axis_name="s"),
    out_type=jax.ShapeDtypeStruct((NC, N_PAD, HALF), jnp.float32),
    scratch_types=[
        pltpu.VMEM((NCHUNK, CH), jnp.int32),
        pltpu.VMEM((NCHUNK, CH), jnp.int32),
        pltpu.VMEM((NCHUNK, CH), jnp.float32),
        pltpu.VMEM((CH, HALF), jnp.float32),
        pltpu.VMEM((CH, HALF), jnp.float32),
        pltpu.VMEM((CH, HALF), jnp.float32),
        pltpu.VMEM((CH, HALF), jnp.float32),
        pltpu.VMEM_SHARED((N_PAD, HALF), jnp.float32),
        pltpu.SemaphoreType.DMA,
        pltpu.SemaphoreType.DMA,
        pltpu.SemaphoreType.DMA,
        pltpu.SemaphoreType.DMA,
    ],
    compiler_params=pltpu.CompilerParams(use_tc_tiling_on_sc=False),
)
def _sc_agg(x_lo, x_hi, ei_h, wgt_h, zero_h, out_h,
            src_v, dst_v, wgt_v, graw0_v, graw1_v, sbuf0_v, sbuf1_v,
            acc_s, gsem0, gsem1, ssem0, ssem1):
    c = lax.axis_index("c")
    s = lax.axis_index("s")

    pltpu.sync_copy(zero_h, acc_s.at[pl.ds(s * ROWS_PER_TILE, ROWS_PER_TILE)])
    pltpu.sync_copy(ei_h.at[0, s], src_v)
    pltpu.sync_copy(ei_h.at[1, s], dst_v)
    pltpu.sync_copy(wgt_h.at[s], wgt_v)
    plsc.subcore_barrier()

    def run_edges(x_h):
        bufs = ((graw0_v, sbuf0_v, gsem0, ssem0),
                (graw1_v, sbuf1_v, gsem1, ssem1))

        pltpu.async_copy(x_h.at[src_v.at[0]], graw0_v, gsem0)
        pltpu.async_copy(x_h.at[src_v.at[1]], graw1_v, gsem1)

        def do_pair(i, carry):
            for b in range(2):
                graw_v, sbuf_v, gsem, ssem = bufs[b]
                k = 2 * i + b
                pltpu.make_async_copy(x_h.at[src_v.at[k]], graw_v, gsem).wait()

                @pl.when(i >= 1)
                def _(sbuf_v=sbuf_v, ssem=ssem):
                    pltpu.make_async_copy(
                        sbuf_v, acc_s.at[dst_v.at[0]], ssem).wait()

                def mul_grp(g, carry2, graw_v=graw_v, sbuf_v=sbuf_v, k=k):
                    wv = wgt_v[k, pl.ds(g * 16, 16)]
                    for t in range(16):
                        e = g * 16 + t
                        w = wv[t]
                        for j in range(HALF // 16):
                            sl = pl.ds(j * 16, 16)
                            sbuf_v[e, sl] = graw_v[e, sl] * w
                    return carry2

                lax.fori_loop(0, CH // 16, mul_grp, 0)

                @pl.when(i < NCHUNK // 2 - 1)
                def _(graw_v=graw_v, gsem=gsem, k=k):
                    pltpu.async_copy(x_h.at[src_v.at[k + 2]], graw_v, gsem)

                pltpu.async_copy(sbuf_v, acc_s.at[dst_v.at[k]], ssem, add=True)

            return carry

        lax.fori_loop(0, NCHUNK // 2, do_pair, 0)

        pltpu.make_async_copy(sbuf0_v, acc_s.at[dst_v.at[0]], ssem0).wait()
        pltpu.make_async_copy(sbuf1_v, acc_s.at[dst_v.at[0]], ssem1).wait()

    @pl.when(c == 0)
    def _():
        run_edges(x_lo)

    @pl.when(c == 1)
    def _():
        run_edges(x_hi)

    plsc.subcore_barrier()
    pltpu.sync_copy(
        acc_s.at[pl.ds(s * ROWS_PER_TILE, ROWS_PER_TILE)],
        out_h.at[c, pl.ds(s * ROWS_PER_TILE, ROWS_PER_TILE)],
    )


def kernel(x, edge_index, edge_weight, W):
    ei4 = edge_index.astype(jnp.int32).reshape(2, NS, NCHUNK, CH)
    x_lo = x[:, :HALF]
    x_hi = x[:, HALF:]
    wgt3 = edge_weight.reshape(NS, NCHUNK, CH)
    zeros = jnp.zeros((ROWS_PER_TILE, HALF), jnp.float32)
    agg2 = _sc_agg(x_lo, x_hi, ei4, wgt3, zeros)
    return _matmul(agg2, W)


# speedup vs baseline: 1.6325x; 1.0149x over previous
import functools

import jax
import jax.numpy as jnp
from jax import lax
from jax.experimental import pallas as pl
from jax.experimental.pallas import tpu as pltpu
from jax.experimental.pallas import tpu_sc as plsc

N_NODES = 10000
N_EDGES = 320000
D_IN = 128
D_OUT = 128
HALF = 64

NC = 2
NS = 16
EDGES_PER_SUBCORE = N_EDGES // NS
CH = 80
NCHUNK = EDGES_PER_SUBCORE // CH
N_PAD = 10240
ROWS_PER_TILE = N_PAD // NS


def _mm_body(a_ref, b_ref, w_ref, o_ref):
    o_ref[...] = (
        jnp.dot(a_ref[0], w_ref[:HALF, :], preferred_element_type=jnp.float32)
        + jnp.dot(b_ref[0], w_ref[HALF:, :], preferred_element_type=jnp.float32)
    )


def _matmul(agg2, W):
    R = 1000
    return pl.pallas_call(
        _mm_body,
        grid=(N_NODES // R,),
        in_specs=[
            pl.BlockSpec((1, R, HALF), lambda i: (0, i, 0)),
            pl.BlockSpec((1, R, HALF), lambda i: (1, i, 0)),
            pl.BlockSpec((D_IN, D_OUT), lambda i: (0, 0)),
        ],
        out_specs=pl.BlockSpec((R, D_OUT), lambda i: (i, 0)),
        out_shape=jax.ShapeDtypeStruct((N_NODES, D_OUT), jnp.float32),
    )(agg2, agg2, W)


@functools.partial(
    pl.kernel,
    mesh=plsc.VectorSubcoreMesh(core_axis_name="c", subcore_axis_name="s"),
    out_type=jax.ShapeDtypeStruct((NC, N_PAD, HALF), jnp.float32),
    scratch_types=[
        pltpu.VMEM((NCHUNK, CH), jnp.int32),
        pltpu.VMEM((NCHUNK, CH), jnp.int32),
        pltpu.VMEM((NCHUNK, CH), jnp.float32),
        pltpu.VMEM((CH, HALF), jnp.float32),
        pltpu.VMEM((CH, HALF), jnp.float32),
        pltpu.VMEM((CH, HALF), jnp.float32),
        pltpu.VMEM((CH, HALF), jnp.float32),
        pltpu.VMEM_SHARED((N_PAD, HALF), jnp.float32),
        pltpu.SemaphoreType.DMA,
        pltpu.SemaphoreType.DMA,
        pltpu.SemaphoreType.DMA,
        pltpu.SemaphoreType.DMA,
    ],
    compiler_params=pltpu.CompilerParams(use_tc_tiling_on_sc=False),
)
def _sc_agg(x_lo, x_hi, ei_h, wgt_h, zero_h, out_h,
            src_v, dst_v, wgt_v, graw0_v, graw1_v, sbuf0_v, sbuf1_v,
            acc_s, gsem0, gsem1, ssem0, ssem1):
    c = lax.axis_index("c")
    s = lax.axis_index("s")

    zc = pltpu.async_copy(
        zero_h, acc_s.at[pl.ds(s * ROWS_PER_TILE, ROWS_PER_TILE)], gsem0)
    sc = pltpu.async_copy(ei_h.at[0, s], src_v, gsem1)
    dc = pltpu.async_copy(ei_h.at[1, s], dst_v, ssem0)
    wc = pltpu.async_copy(wgt_h.at[s], wgt_v, ssem1)
    zc.wait()
    sc.wait()
    dc.wait()
    wc.wait()
    plsc.subcore_barrier()

    def run_edges(x_h):
        bufs = ((graw0_v, sbuf0_v, gsem0, ssem0),
                (graw1_v, sbuf1_v, gsem1, ssem1))

        pltpu.async_copy(x_h.at[src_v.at[0]], graw0_v, gsem0)
        pltpu.async_copy(x_h.at[src_v.at[1]], graw1_v, gsem1)

        def do_pair(i, carry):
            for b in range(2):
                graw_v, sbuf_v, gsem, ssem = bufs[b]
                k = 2 * i + b
                pltpu.make_async_copy(x_h.at[src_v.at[k]], graw_v, gsem).wait()

                @pl.when(i >= 1)
                def _(sbuf_v=sbuf_v, ssem=ssem):
                    pltpu.make_async_copy(
                        sbuf_v, acc_s.at[dst_v.at[0]], ssem).wait()

                def mul_grp(g, carry2, graw_v=graw_v, sbuf_v=sbuf_v, k=k):
                    wv = wgt_v[k, pl.ds(g * 16, 16)]
                    for t in range(16):
                        e = g * 16 + t
                        w = wv[t]
                        for j in range(HALF // 16):
                            sl = pl.ds(j * 16, 16)
                            sbuf_v[e, sl] = graw_v[e, sl] * w
                    return carry2

                lax.fori_loop(0, CH // 16, mul_grp, 0)

                @pl.when(i < NCHUNK // 2 - 1)
                def _(graw_v=graw_v, gsem=gsem, k=k):
                    pltpu.async_copy(x_h.at[src_v.at[k + 2]], graw_v, gsem)

                pltpu.async_copy(sbuf_v, acc_s.at[dst_v.at[k]], ssem, add=True)

            return carry

        lax.fori_loop(0, NCHUNK // 2, do_pair, 0)

        pltpu.make_async_copy(sbuf0_v, acc_s.at[dst_v.at[0]], ssem0).wait()
        pltpu.make_async_copy(sbuf1_v, acc_s.at[dst_v.at[0]], ssem1).wait()

    @pl.when(c == 0)
    def _():
        run_edges(x_lo)

    @pl.when(c == 1)
    def _():
        run_edges(x_hi)

    plsc.subcore_barrier()
    pltpu.sync_copy(
        acc_s.at[pl.ds(s * ROWS_PER_TILE, ROWS_PER_TILE)],
        out_h.at[c, pl.ds(s * ROWS_PER_TILE, ROWS_PER_TILE)],
    )


def kernel(x, edge_index, edge_weight, W):
    ei4 = edge_index.astype(jnp.int32).reshape(2, NS, NCHUNK, CH)
    x_lo = x[:, :HALF]
    x_hi = x[:, HALF:]
    wgt3 = edge_weight.reshape(NS, NCHUNK, CH)
    zeros = jnp.zeros((ROWS_PER_TILE, HALF), jnp.float32)
    agg2 = _sc_agg(x_lo, x_hi, ei4, wgt3, zeros)
    return _matmul(agg2, W)
